# baseline (speedup 1.0000x reference)
import functools

import jax
import jax.numpy as jnp
from jax import lax
from jax.experimental import pallas as pl
from jax.experimental.pallas import tpu as pltpu
from jax.experimental.pallas import tpu_sc as plsc

N = 10000
E = 320000
F_IN = 128
HID = 64
C = 64
W = 128
K = 10

NC = 2
NS = 16
NW = NC * NS
CHUNK = 128
CPT = 80
EPT = CHUNK * CPT
E_PAD = NW * EPT
NP = 10240
RS = NP // NS
DCH = 16
NB = 1024
GRID = NP // NB


def _mlp_body(x_ref, w1_ref, b1_ref, w2_ref, b2_ref, o_ref):
    h = jnp.dot(x_ref[...], w1_ref[...], preferred_element_type=jnp.float32)
    h = jnp.maximum(h + b1_ref[...], 0.0)
    o_ref[:, :C] = (
        jnp.dot(h, w2_ref[...], preferred_element_type=jnp.float32) + b2_ref[...]
    )
    o_ref[:, C:] = jnp.zeros((NB, W - C), jnp.float32)


def _mlp(xp, w1, b1, w2, b2):
    return pl.pallas_call(
        _mlp_body,
        grid=(GRID,),
        in_specs=[
            pl.BlockSpec((NB, F_IN), lambda i: (i, 0)),
            pl.BlockSpec((F_IN, HID), lambda i: (0, 0)),
            pl.BlockSpec((HID,), lambda i: (0,)),
            pl.BlockSpec((HID, C), lambda i: (0, 0)),
            pl.BlockSpec((C,), lambda i: (0,)),
        ],
        out_specs=pl.BlockSpec((NB, W), lambda i: (i, 0)),
        out_shape=jax.ShapeDtypeStruct((NP, W), jnp.float32),
    )(xp, w1, b1, w2, b2)


def _dis_body(degp_ref, dis_ref):
    deg = jnp.sum(degp_ref[...], axis=0, keepdims=True)
    good = deg > 0.0
    dis_ref[...] = jnp.where(good, lax.rsqrt(jnp.where(good, deg, 1.0)), 0.0)


def _dis(degp):
    return pl.pallas_call(
        _dis_body,
        grid=(1,),
        in_specs=[pl.BlockSpec((NC, NP), lambda i: (0, 0))],
        out_specs=pl.BlockSpec((1, NP), lambda i: (0, 0)),
        out_shape=jax.ShapeDtypeStruct((1, NP), jnp.float32),
    )(degp)


def _prep_body(h_ref, dis_ref, coe0_ref, zs_ref, out_ref):
    h = h_ref[...]
    zs_ref[...] = dis_ref[...] * h
    out_ref[...] = coe0_ref[0, 0] * h


def _prep(h, dis2, coe0):
    return pl.pallas_call(
        _prep_body,
        grid=(GRID,),
        in_specs=[
            pl.BlockSpec((NB, W), lambda i: (i, 0)),
            pl.BlockSpec((NB, W), lambda i: (i, 0)),
            pl.BlockSpec((1, 1), lambda i: (0, 0)),
        ],
        out_specs=[
            pl.BlockSpec((NB, W), lambda i: (i, 0)),
            pl.BlockSpec((NB, W), lambda i: (i, 0)),
        ],
        out_shape=[
            jax.ShapeDtypeStruct((NP, W), jnp.float32),
            jax.ShapeDtypeStruct((NP, W), jnp.float32),
        ],
    )(h, dis2, coe0)


def _comb_body(fac, p_ref, tx0_ref, dis_ref, out_ref, coe_ref,
               tx2_ref, outn_ref, zs_ref):
    dis = dis_ref[...]
    pr = -dis * (p_ref[0] + p_ref[1])
    tx2 = fac * pr - tx0_ref[...]
    tx2_ref[...] = tx2
    outn_ref[...] = out_ref[...] + coe_ref[0, 0] * tx2
    zs_ref[...] = dis * tx2


def _combine(p, tx0, dis2, out, coe_i, fac):
    return pl.pallas_call(
        functools.partial(_comb_body, fac),
        grid=(GRID,),
        in_specs=[
            pl.BlockSpec((NC, NB, W), lambda i: (0, i, 0)),
            pl.BlockSpec((NB, W), lambda i: (i, 0)),
            pl.BlockSpec((NB, W), lambda i: (i, 0)),
            pl.BlockSpec((NB, W), lambda i: (i, 0)),
            pl.BlockSpec((1, 1), lambda i: (0, 0)),
        ],
        out_specs=[
            pl.BlockSpec((NB, W), lambda i: (i, 0)),
            pl.BlockSpec((NB, W), lambda i: (i, 0)),
            pl.BlockSpec((NB, W), lambda i: (i, 0)),
        ],
        out_shape=[
            jax.ShapeDtypeStruct((NP, W), jnp.float32),
            jax.ShapeDtypeStruct((NP, W), jnp.float32),
            jax.ShapeDtypeStruct((NP, W), jnp.float32),
        ],
    )(p, tx0, dis2, out, coe_i)


def _last_body(p_ref, tx0_ref, dis_ref, out_ref, coe_ref, fin_ref):
    pr = -dis_ref[...] * (p_ref[0] + p_ref[1])
    tx2 = 2.0 * pr - tx0_ref[...]
    o = (out_ref[...] + coe_ref[0, 0] * tx2)[:, :C]
    m = jnp.max(o, axis=1, keepdims=True)
    e = o - m
    lse = jnp.log(jnp.sum(jnp.exp(e), axis=1, keepdims=True))
    fin_ref[...] = e - lse


def _combine_last(p, tx0, dis2, out, coe_i):
    return pl.pallas_call(
        _last_body,
        grid=(GRID,),
        in_specs=[
            pl.BlockSpec((NC, NB, W), lambda i: (0, i, 0)),
            pl.BlockSpec((NB, W), lambda i: (i, 0)),
            pl.BlockSpec((NB, W), lambda i: (i, 0)),
            pl.BlockSpec((NB, W), lambda i: (i, 0)),
            pl.BlockSpec((1, 1), lambda i: (0, 0)),
        ],
        out_specs=pl.BlockSpec((NB, C), lambda i: (i, 0)),
        out_shape=jax.ShapeDtypeStruct((NP, C), jnp.float32),
    )(p, tx0, dis2, out, coe_i)


def _deg_body(rowp_hbm, zeros1_hbm, degp_hbm, rowv, onesv, degs):
    c = lax.axis_index("c")
    s = lax.axis_index("s")
    wid = c * NS + s
    for k in range(CHUNK // 16):
        onesv[pl.ds(k * 16, 16)] = jnp.ones((16,), jnp.float32)
    pltpu.sync_copy(zeros1_hbm, degs.at[pl.ds(s * RS, RS)])
    plsc.subcore_barrier()

    for r in range(CPT // DCH):
        pltpu.sync_copy(rowp_hbm.at[wid].at[pl.ds(r * DCH, DCH)], rowv)

        def body(j, carry):
            pltpu.sync_copy(onesv, degs.at[rowv.at[j]], add=True)
            return carry

        lax.fori_loop(0, DCH, body, 0)
    plsc.subcore_barrier()
    pltpu.sync_copy(degs.at[pl.ds(s * RS, RS)],
                    degp_hbm.at[c].at[pl.ds(s * RS, RS)])


def _prop_body(colp_hbm, rowp_hbm, zs_hbm, zeros2_hbm, out_hbm,
               colv, rowv, bufa, acc_s, sema):
    c = lax.axis_index("c")
    s = lax.axis_index("s")
    wid = c * NS + s
    pltpu.sync_copy(colp_hbm.at[wid], colv)
    pltpu.sync_copy(rowp_hbm.at[wid], rowv)
    pltpu.sync_copy(zeros2_hbm, acc_s.at[pl.ds(s * RS, RS)])
    plsc.subcore_barrier()

    def body(j, carry):
        pltpu.async_copy(zs_hbm.at[colv.at[j]], bufa, sema).wait()
        pltpu.sync_copy(bufa, acc_s.at[rowv.at[j]], add=True)
        return carry

    lax.fori_loop(0, CPT, body, 0)
    plsc.subcore_barrier()
    pltpu.sync_copy(acc_s.at[pl.ds(s * RS, RS)],
                    out_hbm.at[c].at[pl.ds(s * RS, RS)])


@functools.cache
def _sc_kernels():
    mesh = plsc.VectorSubcoreMesh(
        core_axis_name="c", subcore_axis_name="s",
        num_cores=NC, num_subcores=NS)
    deg = pl.kernel(
        _deg_body,
        out_type=jax.ShapeDtypeStruct((NC, NP), jnp.float32),
        mesh=mesh,
        scratch_types=[
            pltpu.VMEM((DCH, CHUNK), jnp.int32),
            pltpu.VMEM((CHUNK,), jnp.float32),
            pltpu.VMEM_SHARED((NP,), jnp.float32),
        ],
    )
    prop = pl.kernel(
        _prop_body,
        out_type=jax.ShapeDtypeStruct((NC, NP, W), jnp.float32),
        mesh=mesh,
        scratch_types=[
            pltpu.VMEM((CPT, CHUNK), jnp.int32),
            pltpu.VMEM((CPT, CHUNK), jnp.int32),
            pltpu.VMEM((CHUNK, W), jnp.float32),
            pltpu.VMEM_SHARED((NP, W), jnp.float32),
            pltpu.SemaphoreType.DMA,
        ],
    )
    return deg, prop


def kernel(x, edge_index, lin1_w, lin1_b, lin2_w, lin2_b, temp):
    order = jnp.argsort(edge_index[1])
    row = edge_index[0][order]
    col = edge_index[1][order]
    pad = E_PAD - E
    rowp = jnp.concatenate(
        [row, jnp.full((pad,), N, jnp.int32)]).reshape(NW, CPT, CHUNK)
    colp = jnp.concatenate(
        [col, jnp.zeros((pad,), jnp.int32)]).reshape(NW, CPT, CHUNK)
    coe = (temp / (jnp.arange(K + 1, dtype=jnp.float32) + 1.0)).reshape(K + 1, 1, 1)
    zeros1 = jnp.zeros((RS,), jnp.float32)
    zeros2 = jnp.zeros((RS, W), jnp.float32)
    xp = jnp.pad(x, ((0, NP - N), (0, 0)))

    deg_sc, prop_sc = _sc_kernels()
    h = _mlp(xp, lin1_w, lin1_b, lin2_w, lin2_b)
    degp = deg_sc(rowp, zeros1)
    dis_row = _dis(degp)
    dis2 = jnp.broadcast_to(dis_row[0, :, None], (NP, W))
    zs, out = _prep(h, dis2, coe[0])

    tx_prev2 = jnp.zeros((NP, W), jnp.float32)
    tx_prev1 = h
    for i in range(1, K + 1):
        p = prop_sc(colp, rowp, zs, zeros2)
        if i < K:
            fac = 1.0 if i == 1 else 2.0
            tx2, out, zs = _combine(p, tx_prev2, dis2, out, coe[i], fac)
            tx_prev2 = tx_prev1
            tx_prev1 = tx2
        else:
            final = _combine_last(p, tx_prev2, dis2, out, coe[i])
    return final[:N]


# speedup vs baseline: 1.1265x; 1.1265x over previous
import functools

import jax
import jax.numpy as jnp
from jax import lax
from jax.experimental import pallas as pl
from jax.experimental.pallas import tpu as pltpu
from jax.experimental.pallas import tpu_sc as plsc

N = 10000
E = 320000
F_IN = 128
HID = 64
C = 64
W = 128
K = 10

NC = 2
NS = 16
NW = NC * NS
CHUNK = 128
CPT = 80
EPT = CHUNK * CPT
E_PAD = NW * EPT
NP = 10240
RS = NP // NS
DCH = 16
NB = 1024
GRID = NP // NB


def _mlp_body(x_ref, w1_ref, b1_ref, w2_ref, b2_ref, o_ref):
    h = jnp.dot(x_ref[...], w1_ref[...], preferred_element_type=jnp.float32)
    h = jnp.maximum(h + b1_ref[...], 0.0)
    o_ref[:, :C] = (
        jnp.dot(h, w2_ref[...], preferred_element_type=jnp.float32) + b2_ref[...]
    )
    o_ref[:, C:] = jnp.zeros((NB, W - C), jnp.float32)


def _mlp(xp, w1, b1, w2, b2):
    return pl.pallas_call(
        _mlp_body,
        grid=(GRID,),
        in_specs=[
            pl.BlockSpec((NB, F_IN), lambda i: (i, 0)),
            pl.BlockSpec((F_IN, HID), lambda i: (0, 0)),
            pl.BlockSpec((HID,), lambda i: (0,)),
            pl.BlockSpec((HID, C), lambda i: (0, 0)),
            pl.BlockSpec((C,), lambda i: (0,)),
        ],
        out_specs=pl.BlockSpec((NB, W), lambda i: (i, 0)),
        out_shape=jax.ShapeDtypeStruct((NP, W), jnp.float32),
    )(xp, w1, b1, w2, b2)


def _dis_body(degp_ref, dis_ref):
    deg = jnp.sum(degp_ref[...], axis=0, keepdims=True)
    good = deg > 0.0
    dis_ref[...] = jnp.where(good, lax.rsqrt(jnp.where(good, deg, 1.0)), 0.0)


def _dis(degp):
    return pl.pallas_call(
        _dis_body,
        grid=(1,),
        in_specs=[pl.BlockSpec((NC, NP), lambda i: (0, 0))],
        out_specs=pl.BlockSpec((1, NP), lambda i: (0, 0)),
        out_shape=jax.ShapeDtypeStruct((1, NP), jnp.float32),
    )(degp)


def _prep_body(h_ref, dis_ref, coe0_ref, zs_ref, out_ref):
    h = h_ref[...]
    zs_ref[...] = dis_ref[...] * h
    out_ref[...] = coe0_ref[0, 0] * h


def _prep(h, dis2, coe0):
    return pl.pallas_call(
        _prep_body,
        grid=(GRID,),
        in_specs=[
            pl.BlockSpec((NB, W), lambda i: (i, 0)),
            pl.BlockSpec((NB, W), lambda i: (i, 0)),
            pl.BlockSpec((1, 1), lambda i: (0, 0)),
        ],
        out_specs=[
            pl.BlockSpec((NB, W), lambda i: (i, 0)),
            pl.BlockSpec((NB, W), lambda i: (i, 0)),
        ],
        out_shape=[
            jax.ShapeDtypeStruct((NP, W), jnp.float32),
            jax.ShapeDtypeStruct((NP, W), jnp.float32),
        ],
    )(h, dis2, coe0)


def _comb_body(fac, p_ref, tx0_ref, dis_ref, out_ref, coe_ref,
               tx2_ref, outn_ref, zs_ref):
    dis = dis_ref[...]
    pr = -dis * (p_ref[0] + p_ref[1])
    tx2 = fac * pr - tx0_ref[...]
    tx2_ref[...] = tx2
    outn_ref[...] = out_ref[...] + coe_ref[0, 0] * tx2
    zs_ref[...] = dis * tx2


def _combine(p, tx0, dis2, out, coe_i, fac):
    return pl.pallas_call(
        functools.partial(_comb_body, fac),
        grid=(GRID,),
        in_specs=[
            pl.BlockSpec((NC, NB, W), lambda i: (0, i, 0)),
            pl.BlockSpec((NB, W), lambda i: (i, 0)),
            pl.BlockSpec((NB, W), lambda i: (i, 0)),
            pl.BlockSpec((NB, W), lambda i: (i, 0)),
            pl.BlockSpec((1, 1), lambda i: (0, 0)),
        ],
        out_specs=[
            pl.BlockSpec((NB, W), lambda i: (i, 0)),
            pl.BlockSpec((NB, W), lambda i: (i, 0)),
            pl.BlockSpec((NB, W), lambda i: (i, 0)),
        ],
        out_shape=[
            jax.ShapeDtypeStruct((NP, W), jnp.float32),
            jax.ShapeDtypeStruct((NP, W), jnp.float32),
            jax.ShapeDtypeStruct((NP, W), jnp.float32),
        ],
    )(p, tx0, dis2, out, coe_i)


def _last_body(p_ref, tx0_ref, dis_ref, out_ref, coe_ref, fin_ref):
    pr = -dis_ref[...] * (p_ref[0] + p_ref[1])
    tx2 = 2.0 * pr - tx0_ref[...]
    o = (out_ref[...] + coe_ref[0, 0] * tx2)[:, :C]
    m = jnp.max(o, axis=1, keepdims=True)
    e = o - m
    lse = jnp.log(jnp.sum(jnp.exp(e), axis=1, keepdims=True))
    fin_ref[...] = e - lse


def _combine_last(p, tx0, dis2, out, coe_i):
    return pl.pallas_call(
        _last_body,
        grid=(GRID,),
        in_specs=[
            pl.BlockSpec((NC, NB, W), lambda i: (0, i, 0)),
            pl.BlockSpec((NB, W), lambda i: (i, 0)),
            pl.BlockSpec((NB, W), lambda i: (i, 0)),
            pl.BlockSpec((NB, W), lambda i: (i, 0)),
            pl.BlockSpec((1, 1), lambda i: (0, 0)),
        ],
        out_specs=pl.BlockSpec((NB, C), lambda i: (i, 0)),
        out_shape=jax.ShapeDtypeStruct((NP, C), jnp.float32),
    )(p, tx0, dis2, out, coe_i)


def _deg_body(rowp_hbm, zeros1_hbm, degp_hbm, rowv, onesv, degs):
    c = lax.axis_index("c")
    s = lax.axis_index("s")
    wid = c * NS + s
    for k in range(CHUNK // 16):
        onesv[pl.ds(k * 16, 16)] = jnp.ones((16,), jnp.float32)
    pltpu.sync_copy(zeros1_hbm, degs.at[pl.ds(s * RS, RS)])
    plsc.subcore_barrier()

    for r in range(CPT // DCH):
        pltpu.sync_copy(rowp_hbm.at[wid].at[pl.ds(r * DCH, DCH)], rowv)

        def body(j, carry):
            pltpu.sync_copy(onesv, degs.at[rowv.at[j]], add=True)
            return carry

        lax.fori_loop(0, DCH, body, 0)
    plsc.subcore_barrier()
    pltpu.sync_copy(degs.at[pl.ds(s * RS, RS)],
                    degp_hbm.at[c].at[pl.ds(s * RS, RS)])


def _prop_body(colp_hbm, rowp_hbm, zs_hbm, zeros2_hbm, out_hbm,
               colv, rowv, bufa, bufb, acc_s, sema, semb):
    c = lax.axis_index("c")
    s = lax.axis_index("s")
    wid = c * NS + s
    pltpu.sync_copy(zeros2_hbm, acc_s.at[pl.ds(s * RS, RS)])
    plsc.subcore_barrier()

    for r in range(CPT // DCH):
        pltpu.sync_copy(colp_hbm.at[wid].at[pl.ds(r * DCH, DCH)], colv)
        pltpu.sync_copy(rowp_hbm.at[wid].at[pl.ds(r * DCH, DCH)], rowv)

        def body(m, carry):
            j = 2 * m
            da = pltpu.async_copy(zs_hbm.at[colv.at[j]], bufa, sema)
            db = pltpu.async_copy(zs_hbm.at[colv.at[j + 1]], bufb, semb)
            da.wait()
            pltpu.sync_copy(bufa, acc_s.at[rowv.at[j]], add=True)
            db.wait()
            pltpu.sync_copy(bufb, acc_s.at[rowv.at[j + 1]], add=True)
            return carry

        lax.fori_loop(0, DCH // 2, body, 0)
    plsc.subcore_barrier()
    pltpu.sync_copy(acc_s.at[pl.ds(s * RS, RS)],
                    out_hbm.at[c].at[pl.ds(s * RS, RS)])


@functools.cache
def _sc_kernels():
    mesh = plsc.VectorSubcoreMesh(
        core_axis_name="c", subcore_axis_name="s",
        num_cores=NC, num_subcores=NS)
    deg = pl.kernel(
        _deg_body,
        out_type=jax.ShapeDtypeStruct((NC, NP), jnp.float32),
        mesh=mesh,
        scratch_types=[
            pltpu.VMEM((DCH, CHUNK), jnp.int32),
            pltpu.VMEM((CHUNK,), jnp.float32),
            pltpu.VMEM_SHARED((NP,), jnp.float32),
        ],
    )
    prop = pl.kernel(
        _prop_body,
        out_type=jax.ShapeDtypeStruct((NC, NP, W), jnp.float32),
        mesh=mesh,
        scratch_types=[
            pltpu.VMEM((DCH, CHUNK), jnp.int32),
            pltpu.VMEM((DCH, CHUNK), jnp.int32),
            pltpu.VMEM((CHUNK, W), jnp.float32),
            pltpu.VMEM((CHUNK, W), jnp.float32),
            pltpu.VMEM_SHARED((NP, W), jnp.float32),
            pltpu.SemaphoreType.DMA,
            pltpu.SemaphoreType.DMA,
        ],
    )
    return deg, prop


def kernel(x, edge_index, lin1_w, lin1_b, lin2_w, lin2_b, temp):
    row = edge_index[0]
    col = edge_index[1]
    pad = E_PAD - E
    rowp = jnp.concatenate(
        [row, jnp.full((pad,), N, jnp.int32)]).reshape(NW, CPT, CHUNK)
    colp = jnp.concatenate(
        [col, jnp.zeros((pad,), jnp.int32)]).reshape(NW, CPT, CHUNK)
    coe = (temp / (jnp.arange(K + 1, dtype=jnp.float32) + 1.0)).reshape(K + 1, 1, 1)
    zeros1 = jnp.zeros((RS,), jnp.float32)
    zeros2 = jnp.zeros((RS, W), jnp.float32)
    xp = jnp.pad(x, ((0, NP - N), (0, 0)))

    deg_sc, prop_sc = _sc_kernels()
    h = _mlp(xp, lin1_w, lin1_b, lin2_w, lin2_b)
    degp = deg_sc(rowp, zeros1)
    dis_row = _dis(degp)
    dis2 = jnp.broadcast_to(dis_row[0, :, None], (NP, W))
    zs, out = _prep(h, dis2, coe[0])

    tx_prev2 = jnp.zeros((NP, W), jnp.float32)
    tx_prev1 = h
    for i in range(1, K + 1):
        p = prop_sc(colp, rowp, zs, zeros2)
        if i < K:
            fac = 1.0 if i == 1 else 2.0
            tx2, out, zs = _combine(p, tx_prev2, dis2, out, coe[i], fac)
            tx_prev2 = tx_prev1
            tx_prev1 = tx2
        else:
            final = _combine_last(p, tx_prev2, dis2, out, coe[i])
    return final[:N]


# speedup vs baseline: 1.1309x; 1.0039x over previous
import functools

import jax
import jax.numpy as jnp
from jax import lax
from jax.experimental import pallas as pl
from jax.experimental.pallas import tpu as pltpu
from jax.experimental.pallas import tpu_sc as plsc

N = 10000
E = 320000
F_IN = 128
HID = 64
C = 64
W = 128
K = 10

NC = 2
NS = 16
NW = NC * NS
CHUNK = 128
CPT = 80
EPT = CHUNK * CPT
E_PAD = NW * EPT
NP = 10240
RS = NP // NS
DCH = 16
NB = 1024
GRID = NP // NB


def _mlp_body(x_ref, w1_ref, b1_ref, w2_ref, b2_ref, o_ref):
    h = jnp.dot(x_ref[...], w1_ref[...], preferred_element_type=jnp.float32)
    h = jnp.maximum(h + b1_ref[...], 0.0)
    o_ref[:, :C] = (
        jnp.dot(h, w2_ref[...], preferred_element_type=jnp.float32) + b2_ref[...]
    )
    o_ref[:, C:] = jnp.zeros((NB, W - C), jnp.float32)


def _mlp(xp, w1, b1, w2, b2):
    return pl.pallas_call(
        _mlp_body,
        grid=(GRID,),
        in_specs=[
            pl.BlockSpec((NB, F_IN), lambda i: (i, 0)),
            pl.BlockSpec((F_IN, HID), lambda i: (0, 0)),
            pl.BlockSpec((HID,), lambda i: (0,)),
            pl.BlockSpec((HID, C), lambda i: (0, 0)),
            pl.BlockSpec((C,), lambda i: (0,)),
        ],
        out_specs=pl.BlockSpec((NB, W), lambda i: (i, 0)),
        out_shape=jax.ShapeDtypeStruct((NP, W), jnp.float32),
    )(xp, w1, b1, w2, b2)


def _dis_body(degp_ref, dis_ref):
    deg = jnp.sum(degp_ref[...], axis=0, keepdims=True)
    good = deg > 0.0
    dis_ref[...] = jnp.where(good, lax.rsqrt(jnp.where(good, deg, 1.0)), 0.0)


def _dis(degp):
    return pl.pallas_call(
        _dis_body,
        grid=(1,),
        in_specs=[pl.BlockSpec((NC, NP), lambda i: (0, 0))],
        out_specs=pl.BlockSpec((1, NP), lambda i: (0, 0)),
        out_shape=jax.ShapeDtypeStruct((1, NP), jnp.float32),
    )(degp)


def _prep_body(h_ref, dis_ref, coe0_ref, zs_ref, out_ref):
    h = h_ref[...]
    zs_ref[...] = dis_ref[...] * h
    out_ref[...] = coe0_ref[0, 0] * h


def _prep(h, dis2, coe0):
    return pl.pallas_call(
        _prep_body,
        grid=(GRID,),
        in_specs=[
            pl.BlockSpec((NB, W), lambda i: (i, 0)),
            pl.BlockSpec((NB, W), lambda i: (i, 0)),
            pl.BlockSpec((1, 1), lambda i: (0, 0)),
        ],
        out_specs=[
            pl.BlockSpec((NB, W), lambda i: (i, 0)),
            pl.BlockSpec((NB, W), lambda i: (i, 0)),
        ],
        out_shape=[
            jax.ShapeDtypeStruct((NP, W), jnp.float32),
            jax.ShapeDtypeStruct((NP, W), jnp.float32),
        ],
    )(h, dis2, coe0)


def _comb_body(fac, p_ref, tx0_ref, dis_ref, out_ref, coe_ref,
               tx2_ref, outn_ref, zs_ref):
    dis = dis_ref[...]
    pr = -dis * (p_ref[0] + p_ref[1])
    tx2 = fac * pr - tx0_ref[...]
    tx2_ref[...] = tx2
    outn_ref[...] = out_ref[...] + coe_ref[0, 0] * tx2
    zs_ref[...] = dis * tx2


def _combine(p, tx0, dis2, out, coe_i, fac):
    return pl.pallas_call(
        functools.partial(_comb_body, fac),
        grid=(GRID,),
        in_specs=[
            pl.BlockSpec((NC, NB, W), lambda i: (0, i, 0)),
            pl.BlockSpec((NB, W), lambda i: (i, 0)),
            pl.BlockSpec((NB, W), lambda i: (i, 0)),
            pl.BlockSpec((NB, W), lambda i: (i, 0)),
            pl.BlockSpec((1, 1), lambda i: (0, 0)),
        ],
        out_specs=[
            pl.BlockSpec((NB, W), lambda i: (i, 0)),
            pl.BlockSpec((NB, W), lambda i: (i, 0)),
            pl.BlockSpec((NB, W), lambda i: (i, 0)),
        ],
        out_shape=[
            jax.ShapeDtypeStruct((NP, W), jnp.float32),
            jax.ShapeDtypeStruct((NP, W), jnp.float32),
            jax.ShapeDtypeStruct((NP, W), jnp.float32),
        ],
    )(p, tx0, dis2, out, coe_i)


def _last_body(p_ref, tx0_ref, dis_ref, out_ref, coe_ref, fin_ref):
    pr = -dis_ref[...] * (p_ref[0] + p_ref[1])
    tx2 = 2.0 * pr - tx0_ref[...]
    o = (out_ref[...] + coe_ref[0, 0] * tx2)[:, :C]
    m = jnp.max(o, axis=1, keepdims=True)
    e = o - m
    lse = jnp.log(jnp.sum(jnp.exp(e), axis=1, keepdims=True))
    fin_ref[...] = e - lse


def _combine_last(p, tx0, dis2, out, coe_i):
    return pl.pallas_call(
        _last_body,
        grid=(GRID,),
        in_specs=[
            pl.BlockSpec((NC, NB, W), lambda i: (0, i, 0)),
            pl.BlockSpec((NB, W), lambda i: (i, 0)),
            pl.BlockSpec((NB, W), lambda i: (i, 0)),
            pl.BlockSpec((NB, W), lambda i: (i, 0)),
            pl.BlockSpec((1, 1), lambda i: (0, 0)),
        ],
        out_specs=pl.BlockSpec((NB, C), lambda i: (i, 0)),
        out_shape=jax.ShapeDtypeStruct((NP, C), jnp.float32),
    )(p, tx0, dis2, out, coe_i)


def _deg_body(rowp_hbm, zeros1_hbm, degp_hbm, rowv, onesv, degs):
    c = lax.axis_index("c")
    s = lax.axis_index("s")
    wid = c * NS + s
    for k in range(CHUNK // 16):
        onesv[pl.ds(k * 16, 16)] = jnp.ones((16,), jnp.float32)
    pltpu.sync_copy(zeros1_hbm, degs.at[pl.ds(s * RS, RS)])
    plsc.subcore_barrier()

    for r in range(CPT // DCH):
        pltpu.sync_copy(rowp_hbm.at[wid].at[pl.ds(r * DCH, DCH)], rowv)

        def body(j, carry):
            pltpu.sync_copy(onesv, degs.at[rowv.at[j]], add=True)
            return carry

        lax.fori_loop(0, DCH, body, 0)
    plsc.subcore_barrier()
    pltpu.sync_copy(degs.at[pl.ds(s * RS, RS)],
                    degp_hbm.at[c].at[pl.ds(s * RS, RS)])


def _prop_body(colp_hbm, rowp_hbm, zs_hbm, zeros2_hbm, out_hbm,
               colv, rowv, bufa, bufb, acc_s, sema, semb, sems):
    c = lax.axis_index("c")
    s = lax.axis_index("s")
    wid = c * NS + s
    pltpu.sync_copy(zeros2_hbm, acc_s.at[pl.ds(s * RS, RS)])
    plsc.subcore_barrier()

    for r in range(CPT // DCH):
        pltpu.sync_copy(colp_hbm.at[wid].at[pl.ds(r * DCH, DCH)], colv)
        pltpu.sync_copy(rowp_hbm.at[wid].at[pl.ds(r * DCH, DCH)], rowv)

        def body(m, carry):
            j = 2 * m
            da = pltpu.async_copy(zs_hbm.at[colv.at[j]], bufa, sema)
            db = pltpu.async_copy(zs_hbm.at[colv.at[j + 1]], bufb, semb)
            da.wait()
            sa = pltpu.async_copy(bufa, acc_s.at[rowv.at[j]], sems, add=True)
            db.wait()
            sb = pltpu.async_copy(bufb, acc_s.at[rowv.at[j + 1]], sems,
                                  add=True)
            sa.wait()
            sb.wait()
            return carry

        lax.fori_loop(0, DCH // 2, body, 0)
    plsc.subcore_barrier()
    pltpu.sync_copy(acc_s.at[pl.ds(s * RS, RS)],
                    out_hbm.at[c].at[pl.ds(s * RS, RS)])


@functools.cache
def _sc_kernels():
    mesh = plsc.VectorSubcoreMesh(
        core_axis_name="c", subcore_axis_name="s",
        num_cores=NC, num_subcores=NS)
    deg = pl.kernel(
        _deg_body,
        out_type=jax.ShapeDtypeStruct((NC, NP), jnp.float32),
        mesh=mesh,
        scratch_types=[
            pltpu.VMEM((DCH, CHUNK), jnp.int32),
            pltpu.VMEM((CHUNK,), jnp.float32),
            pltpu.VMEM_SHARED((NP,), jnp.float32),
        ],
    )
    prop = pl.kernel(
        _prop_body,
        out_type=jax.ShapeDtypeStruct((NC, NP, W), jnp.float32),
        mesh=mesh,
        scratch_types=[
            pltpu.VMEM((DCH, CHUNK), jnp.int32),
            pltpu.VMEM((DCH, CHUNK), jnp.int32),
            pltpu.VMEM((CHUNK, W), jnp.float32),
            pltpu.VMEM((CHUNK, W), jnp.float32),
            pltpu.VMEM_SHARED((NP, W), jnp.float32),
            pltpu.SemaphoreType.DMA,
            pltpu.SemaphoreType.DMA,
            pltpu.SemaphoreType.DMA,
        ],
    )
    return deg, prop


def kernel(x, edge_index, lin1_w, lin1_b, lin2_w, lin2_b, temp):
    row = edge_index[0]
    col = edge_index[1]
    pad = E_PAD - E
    rowp = jnp.concatenate(
        [row, jnp.full((pad,), N, jnp.int32)]).reshape(NW, CPT, CHUNK)
    colp = jnp.concatenate(
        [col, jnp.zeros((pad,), jnp.int32)]).reshape(NW, CPT, CHUNK)
    coe = (temp / (jnp.arange(K + 1, dtype=jnp.float32) + 1.0)).reshape(K + 1, 1, 1)
    zeros1 = jnp.zeros((RS,), jnp.float32)
    zeros2 = jnp.zeros((RS, W), jnp.float32)
    xp = jnp.pad(x, ((0, NP - N), (0, 0)))

    deg_sc, prop_sc = _sc_kernels()
    h = _mlp(xp, lin1_w, lin1_b, lin2_w, lin2_b)
    degp = deg_sc(rowp, zeros1)
    dis_row = _dis(degp)
    dis2 = jnp.broadcast_to(dis_row[0, :, None], (NP, W))
    zs, out = _prep(h, dis2, coe[0])

    tx_prev2 = jnp.zeros((NP, W), jnp.float32)
    tx_prev1 = h
    for i in range(1, K + 1):
        p = prop_sc(colp, rowp, zs, zeros2)
        if i < K:
            fac = 1.0 if i == 1 else 2.0
            tx2, out, zs = _combine(p, tx_prev2, dis2, out, coe[i], fac)
            tx_prev2 = tx_prev1
            tx_prev1 = tx2
        else:
            final = _combine_last(p, tx_prev2, dis2, out, coe[i])
    return final[:N]
